# baseline (speedup 1.0000x reference)
import jax
import jax.numpy as jnp
from jax.experimental import pallas as pl
from jax.experimental.pallas import tpu as pltpu

N_V = 2048
N_E = 4096
D = 256
TJ = 256
NJ = N_E // TJ
PA = NJ


def _normalize_block(m_ref, adj, s):
    r0 = s * TJ
    eye = (jax.lax.broadcasted_iota(jnp.int32, (TJ, TJ), 0) ==
           jax.lax.broadcasted_iota(jnp.int32, (TJ, TJ), 1))
    sub = m_ref[pl.ds(r0, TJ), :]
    m_ref[pl.ds(r0, TJ), :] = jnp.where(eye, jnp.bfloat16(1.0), sub)
    a = m_ref[...].astype(jnp.float32) * adj
    cmax = jnp.max(a, axis=0, keepdims=True)
    m_ref[...] = (a * (1.0 / cmax)).astype(jnp.bfloat16)


def _body(hv_ref, ef_ref, adj_ref, t_ref, w_ref, bias_ref, p_ref,
          out_ref, hp_ref, tsc_ref, ew_ref, m0_ref, m1_ref, m2_ref):
    g = pl.program_id(0)

    @pl.when(g == 0)
    def _():
        hp_ref[...] = jnp.dot(hv_ref[...], p_ref[...],
                              preferred_element_type=jnp.float32)

    @pl.when(g < PA)
    def _():
        tsc_ref[pl.ds(g * TJ, TJ), :] = jnp.transpose(
            (t_ref[...] * hp_ref[...]).astype(jnp.bfloat16))
        ew_ref[pl.ds(g * TJ, TJ), :] = jnp.dot(
            ef_ref[...].astype(jnp.bfloat16), w_ref[...].astype(jnp.bfloat16),
            preferred_element_type=jnp.float32).astype(jnp.bfloat16)

    s = g - PA - 1
    ss = g - PA
    in_b = (g >= PA) & (g < PA + NJ)
    bufs = (m0_ref, m1_ref, m2_ref)

    for r in range(3):
        @pl.when(in_b & (ss % 3 == r))
        def _(r=r):
            bufs[r][...] = jnp.dot(
                tsc_ref[...], t_ref[...].astype(jnp.bfloat16),
                preferred_element_type=jnp.float32).astype(jnp.bfloat16)

    for r in range(3):
        @pl.when((g > PA) & (s % 3 == r))
        def _(r=r):
            _normalize_block(bufs[r], adj_ref[...], s)

            @pl.when(s % 2 == 1)
            def _():
                contrib = (
                    jnp.dot(bufs[(r + 2) % 3][...],
                            ew_ref[pl.ds((s - 1) * TJ, TJ), :],
                            preferred_element_type=jnp.float32) +
                    jnp.dot(bufs[r][...], ew_ref[pl.ds(s * TJ, TJ), :],
                            preferred_element_type=jnp.float32))

                @pl.when(s == 1)
                def _():
                    out_ref[...] = contrib + bias_ref[...]

                @pl.when(s > 1)
                def _():
                    out_ref[...] += contrib


def kernel(H_v, edge_features, adj_e, adj_v, T, weight, bias, p):
    del adj_v

    clip = lambda v, hi: jnp.clip(v, 0, hi)
    return pl.pallas_call(
        _body,
        grid=(PA + NJ + 1,),
        in_specs=[
            pl.BlockSpec((N_V, D), lambda g: (0, 0)),
            pl.BlockSpec((TJ, D), lambda g: (clip(g, PA - 1), 0)),
            pl.BlockSpec((N_E, TJ),
                         lambda g: (0, clip(g - PA - 1, NJ - 1))),
            pl.BlockSpec((N_V, TJ),
                         lambda g: (0, jnp.where(g < PA, g,
                                                 clip(g - PA, NJ - 1)))),
            pl.BlockSpec((D, D), lambda g: (0, 0)),
            pl.BlockSpec((1, D), lambda g: (0, 0)),
            pl.BlockSpec((D, 1), lambda g: (0, 0)),
        ],
        out_specs=pl.BlockSpec((N_E, D), lambda g: (0, 0)),
        out_shape=jax.ShapeDtypeStruct((N_E, D), jnp.float32),
        scratch_shapes=[
            pltpu.VMEM((N_V, 1), jnp.float32),
            pltpu.VMEM((N_E, N_V), jnp.bfloat16),
            pltpu.VMEM((N_E, D), jnp.bfloat16),
            pltpu.VMEM((N_E, TJ), jnp.bfloat16),
            pltpu.VMEM((N_E, TJ), jnp.bfloat16),
            pltpu.VMEM((N_E, TJ), jnp.bfloat16),
        ],
        compiler_params=pltpu.CompilerParams(
            vmem_limit_bytes=63 * 1024 * 1024),
    )(H_v, edge_features, adj_e, T, weight, bias.reshape(1, D),
      p.reshape(D, 1))


# speedup vs baseline: 1.0299x; 1.0299x over previous
import jax
import jax.numpy as jnp
from jax.experimental import pallas as pl
from jax.experimental.pallas import tpu as pltpu

N_V = 2048
N_E = 4096
D = 256
TJ = 256
NJ = N_E // TJ
PA = NJ


def _body(hv_ref, ef_ref, adj_ref, t_ref, w_ref, bias_ref, p_ref,
          out_ref, hp_ref, tsc_ref, ew_ref, mult_ref):
    g = pl.program_id(0)

    @pl.when(g == 0)
    def _():
        hp_ref[...] = jnp.dot(hv_ref[...], p_ref[...],
                              preferred_element_type=jnp.float32)

    @pl.when(g < PA)
    def _():
        tsc_ref[pl.ds(g * TJ, TJ), :] = jnp.transpose(
            (t_ref[...] * hp_ref[...]).astype(jnp.bfloat16))
        ew_ref[pl.ds(g * TJ, TJ), :] = jnp.dot(
            ef_ref[...].astype(jnp.bfloat16), w_ref[...].astype(jnp.bfloat16),
            preferred_element_type=jnp.float32).astype(jnp.bfloat16)

    @pl.when(g > PA)
    def _():
        s = g - PA - 1
        half = (s % 2) * TJ
        r0 = s * TJ
        eye = (jax.lax.broadcasted_iota(jnp.int32, (TJ, TJ), 0) ==
               jax.lax.broadcasted_iota(jnp.int32, (TJ, TJ), 1))
        sub = mult_ref[pl.ds(r0, TJ), pl.ds(half, TJ)]
        mult_ref[pl.ds(r0, TJ), pl.ds(half, TJ)] = jnp.where(
            eye, jnp.bfloat16(1.0), sub)
        a = (mult_ref[:, pl.ds(half, TJ)].astype(jnp.float32) *
             adj_ref[...])
        cmax = jnp.max(a, axis=0, keepdims=True)
        mult_ref[:, pl.ds(half, TJ)] = (
            a * (1.0 / cmax)).astype(jnp.bfloat16)

        @pl.when(s % 2 == 1)
        def _():
            contrib = jnp.dot(mult_ref[...],
                              ew_ref[pl.ds((s - 1) * TJ, 2 * TJ), :],
                              preferred_element_type=jnp.float32)

            @pl.when(s == 1)
            def _():
                out_ref[...] = contrib + bias_ref[...]

            @pl.when(s > 1)
            def _():
                out_ref[...] += contrib

    @pl.when((g >= PA) & (g < PA + NJ))
    def _():
        ss = g - PA
        mult_ref[:, pl.ds((ss % 2) * TJ, TJ)] = jnp.dot(
            tsc_ref[...], t_ref[...].astype(jnp.bfloat16),
            preferred_element_type=jnp.float32).astype(jnp.bfloat16)


def kernel(H_v, edge_features, adj_e, adj_v, T, weight, bias, p):
    del adj_v

    clip = lambda v, hi: jnp.clip(v, 0, hi)
    return pl.pallas_call(
        _body,
        grid=(PA + NJ + 1,),
        in_specs=[
            pl.BlockSpec((N_V, D), lambda g: (0, 0)),
            pl.BlockSpec((TJ, D), lambda g: (clip(g, PA - 1), 0)),
            pl.BlockSpec((N_E, TJ),
                         lambda g: (0, clip(g - PA - 1, NJ - 1))),
            pl.BlockSpec((N_V, TJ),
                         lambda g: (0, jnp.where(g < PA, g,
                                                 clip(g - PA, NJ - 1)))),
            pl.BlockSpec((D, D), lambda g: (0, 0)),
            pl.BlockSpec((1, D), lambda g: (0, 0)),
            pl.BlockSpec((D, 1), lambda g: (0, 0)),
        ],
        out_specs=pl.BlockSpec((N_E, D), lambda g: (0, 0)),
        out_shape=jax.ShapeDtypeStruct((N_E, D), jnp.float32),
        scratch_shapes=[
            pltpu.VMEM((N_V, 1), jnp.float32),
            pltpu.VMEM((N_E, N_V), jnp.bfloat16),
            pltpu.VMEM((N_E, D), jnp.bfloat16),
            pltpu.VMEM((N_E, 2 * TJ), jnp.bfloat16),
        ],
        compiler_params=pltpu.CompilerParams(
            vmem_limit_bytes=63 * 1024 * 1024),
    )(H_v, edge_features, adj_e, T, weight, bias.reshape(1, D),
      p.reshape(D, 1))
